# baseline (speedup 1.0000x reference)
import functools

import jax
import jax.numpy as jnp
from jax import lax
from jax.experimental import pallas as pl
from jax.experimental.pallas import tpu as pltpu
from jax.experimental.pallas import tpu_sc as plsc

N_ATOMS = 100000
M_NBRS = 64
ROW = 16
NC, NS, LANES = 2, 16, 16
NWORKERS = NC * NS
ATOMS_PER_W = 3200
CHUNK_ATOMS = 128
CHUNKS = ATOMS_PER_W // CHUNK_ATOMS
CHUNK_IDX = CHUNK_ATOMS * M_NBRS
HALF_ATOMS = 32
HALF_IDX = HALF_ATOMS * M_NBRS
HALVES = CHUNK_ATOMS // HALF_ATOMS
GATHER = 2048
GATHERS_PER_HALF = HALF_IDX // GATHER
WIN = CHUNK_ATOMS + 8

_mesh = plsc.VectorSubcoreMesh(core_axis_name="c", subcore_axis_name="s")


@functools.partial(
    pl.kernel,
    out_type=jax.ShapeDtypeStruct((N_ATOMS * M_NBRS,), jnp.float32),
    mesh=_mesh,
    scratch_types=[
        pltpu.VMEM((CHUNK_IDX,), jnp.int32),
        pltpu.VMEM((WIN, ROW), jnp.float32),
        [pltpu.VMEM((HALF_IDX, ROW), jnp.float32) for _ in range(2)],
        pltpu.VMEM((CHUNK_IDX,), jnp.float32),
        [pltpu.SemaphoreType.DMA for _ in range(2)],
    ],
    compiler_params=pltpu.CompilerParams(
        use_tc_tiling_on_sc=False, needs_layout_passes=False),
)
def _sc_dist(coords_hbm, nbr_hbm, out_hbm, idx_v, own_v, rbufs, out_v, sems):
    wid = lax.axis_index("s") * NC + lax.axis_index("c")
    start_atom = (wid * (N_ATOMS - ATOMS_PER_W)) // (NWORKERS - 1)

    iota = lax.iota(jnp.int32, LANES)
    col_x = jnp.zeros((LANES,), jnp.int32)
    col_y = col_x + 1
    col_z = col_x + 2

    def fire_half(h, buf, sem):
        for g in range(GATHERS_PER_HALF):
            pltpu.async_copy(
                coords_hbm.at[idx_v.at[pl.ds(h * HALF_IDX + g * GATHER,
                                             GATHER)]],
                buf.at[pl.ds(g * GATHER, GATHER)], sem)

    def drain(buf, sem):
        pltpu.make_async_copy(
            coords_hbm.at[pl.ds(0, HALF_IDX)], buf, sem).wait()

    def compute_half(h, buf, off):
        @pl.loop(0, HALF_ATOMS)
        def _atom(a):
            own_row = own_v[off + h * HALF_ATOMS + a]
            ox = jnp.broadcast_to(own_row[0], (LANES,))
            oy = jnp.broadcast_to(own_row[1], (LANES,))
            oz = jnp.broadcast_to(own_row[2], (LANES,))
            for j in range(M_NBRS // LANES):
                ridx = iota + (a * M_NBRS + j * LANES)
                dx = plsc.load_gather(buf, [ridx, col_x]) - ox
                dy = plsc.load_gather(buf, [ridx, col_y]) - oy
                dz = plsc.load_gather(buf, [ridx, col_z]) - oz
                out_v[pl.ds(h * HALF_IDX + a * M_NBRS + j * LANES, LANES)] = (
                    dx * dx + dy * dy + dz * dz)

    @pl.loop(0, CHUNKS)
    def _chunk(c):
        base_atom = start_atom + c * CHUNK_ATOMS
        base_idx = base_atom * M_NBRS
        pltpu.sync_copy(nbr_hbm.at[pl.ds(base_idx, CHUNK_IDX)], idx_v)
        aligned_base = (base_atom // 8) * 8
        off = base_atom - aligned_base
        pltpu.sync_copy(coords_hbm.at[pl.ds(aligned_base, WIN)], own_v)

        fire_half(0, rbufs[0], sems[0])
        for h in range(HALVES):
            b = h % 2
            if h + 1 < HALVES:
                fire_half(h + 1, rbufs[1 - b], sems[1 - b])
            drain(rbufs[b], sems[b])
            compute_half(h, rbufs[b], off)

        pltpu.sync_copy(out_v, out_hbm.at[pl.ds(base_idx, CHUNK_IDX)])


def kernel(coords, nbr_list):
    coords16 = jnp.pad(coords.astype(jnp.float32), ((0, 8), (0, ROW - 3)))
    nbr = nbr_list.astype(jnp.int32).reshape(-1)
    out = _sc_dist(coords16, nbr)
    return out.reshape(N_ATOMS, M_NBRS)


# speedup vs baseline: 1.1664x; 1.1664x over previous
import functools

import jax
import jax.numpy as jnp
from jax import lax
from jax.experimental import pallas as pl
from jax.experimental.pallas import tpu as pltpu
from jax.experimental.pallas import tpu_sc as plsc

N_ATOMS = 100000
M_NBRS = 64
ROW = 16
NC, NS, LANES = 2, 16, 16
NWORKERS = NC * NS
ATOMS_PER_W = 3200
CHUNK_ATOMS = 64
CHUNKS = ATOMS_PER_W // CHUNK_ATOMS
CHUNK_IDX = CHUNK_ATOMS * M_NBRS
HALF_ATOMS = 8
HALF_IDX = HALF_ATOMS * M_NBRS
HALVES = CHUNK_ATOMS // HALF_ATOMS
GATHER = 512
GATHERS_PER_HALF = HALF_IDX // GATHER
WIN = CHUNK_ATOMS + 8
STAGE_ROWS = 6256
TABLE_ROWS = STAGE_ROWS * NS

_mesh = plsc.VectorSubcoreMesh(core_axis_name="c", subcore_axis_name="s")


@functools.partial(
    pl.kernel,
    out_type=jax.ShapeDtypeStruct((N_ATOMS * M_NBRS,), jnp.float32),
    mesh=_mesh,
    scratch_types=[
        pltpu.VMEM((CHUNK_IDX,), jnp.int32),
        pltpu.VMEM((WIN, ROW), jnp.float32),
        [pltpu.VMEM((HALF_IDX, ROW), jnp.float32) for _ in range(2)],
        pltpu.VMEM((CHUNK_IDX,), jnp.float32),
        [pltpu.SemaphoreType.DMA for _ in range(2)],
        pltpu.VMEM_SHARED((TABLE_ROWS, ROW), jnp.float32),
    ],
    compiler_params=pltpu.CompilerParams(
        use_tc_tiling_on_sc=False, needs_layout_passes=False),
)
def _sc_dist(coords_hbm, nbr_hbm, out_hbm, idx_v, own_v, rbufs, out_v, sems,
             table_sh):
    sid = lax.axis_index("s")
    wid = sid * NC + lax.axis_index("c")
    pltpu.sync_copy(coords_hbm.at[pl.ds(sid * STAGE_ROWS, STAGE_ROWS)],
                    table_sh.at[pl.ds(sid * STAGE_ROWS, STAGE_ROWS)])
    plsc.subcore_barrier()
    start_atom = (wid * (N_ATOMS - ATOMS_PER_W)) // (NWORKERS - 1)

    iota = lax.iota(jnp.int32, LANES)
    col_x = jnp.zeros((LANES,), jnp.int32)
    col_y = col_x + 1
    col_z = col_x + 2

    def fire_half(h, buf, sem):
        for g in range(GATHERS_PER_HALF):
            pltpu.async_copy(
                table_sh.at[idx_v.at[pl.ds(h * HALF_IDX + g * GATHER,
                                           GATHER)]],
                buf.at[pl.ds(g * GATHER, GATHER)], sem)

    def drain(buf, sem):
        pltpu.make_async_copy(
            coords_hbm.at[pl.ds(0, HALF_IDX)], buf, sem).wait()

    def compute_half(h, buf, off):
        @pl.loop(0, HALF_ATOMS)
        def _atom(a):
            own_row = own_v[off + h * HALF_ATOMS + a]
            ox = jnp.broadcast_to(own_row[0], (LANES,))
            oy = jnp.broadcast_to(own_row[1], (LANES,))
            oz = jnp.broadcast_to(own_row[2], (LANES,))
            for j in range(M_NBRS // LANES):
                ridx = iota + (a * M_NBRS + j * LANES)
                dx = plsc.load_gather(buf, [ridx, col_x]) - ox
                dy = plsc.load_gather(buf, [ridx, col_y]) - oy
                dz = plsc.load_gather(buf, [ridx, col_z]) - oz
                out_v[pl.ds(h * HALF_IDX + a * M_NBRS + j * LANES, LANES)] = (
                    dx * dx + dy * dy + dz * dz)

    @pl.loop(0, CHUNKS)
    def _chunk(c):
        base_atom = start_atom + c * CHUNK_ATOMS
        base_idx = base_atom * M_NBRS
        pltpu.sync_copy(nbr_hbm.at[pl.ds(base_idx, CHUNK_IDX)], idx_v)
        aligned_base = (base_atom // 8) * 8
        off = base_atom - aligned_base
        pltpu.sync_copy(table_sh.at[pl.ds(aligned_base, WIN)], own_v)

        fire_half(0, rbufs[0], sems[0])
        for h in range(HALVES):
            b = h % 2
            if h + 1 < HALVES:
                fire_half(h + 1, rbufs[1 - b], sems[1 - b])
            drain(rbufs[b], sems[b])
            compute_half(h, rbufs[b], off)

        pltpu.sync_copy(out_v, out_hbm.at[pl.ds(base_idx, CHUNK_IDX)])


def kernel(coords, nbr_list):
    coords16 = jnp.pad(coords.astype(jnp.float32),
                       ((0, TABLE_ROWS - N_ATOMS), (0, ROW - 3)))
    nbr = nbr_list.astype(jnp.int32).reshape(-1)
    out = _sc_dist(coords16, nbr)
    return out.reshape(N_ATOMS, M_NBRS)


# speedup vs baseline: 1.3708x; 1.1752x over previous
import functools

import jax
import jax.numpy as jnp
from jax import lax
from jax.experimental import pallas as pl
from jax.experimental.pallas import tpu as pltpu
from jax.experimental.pallas import tpu_sc as plsc

N_ATOMS = 100000
M_NBRS = 64
ROW = 16
NC, NS, LANES = 2, 16, 16
NWORKERS = NC * NS
ATOMS_PER_W = 3200
CHUNK_ATOMS = 32
CHUNKS = ATOMS_PER_W // CHUNK_ATOMS
CHUNK_IDX = CHUNK_ATOMS * M_NBRS
HALF_ATOMS = 8
HALF_IDX = HALF_ATOMS * M_NBRS
HALVES = CHUNK_ATOMS // HALF_ATOMS
WIN = CHUNK_ATOMS + 8
STAGE_ROWS = 6256
TABLE_ROWS = STAGE_ROWS * NS

_mesh = plsc.VectorSubcoreMesh(core_axis_name="c", subcore_axis_name="s")


@functools.partial(
    pl.kernel,
    out_type=jax.ShapeDtypeStruct((N_ATOMS * M_NBRS,), jnp.float32),
    mesh=_mesh,
    scratch_types=[
        [pltpu.VMEM((CHUNK_IDX,), jnp.int32) for _ in range(2)],
        pltpu.VMEM((WIN, ROW), jnp.float32),
        [pltpu.VMEM((HALF_IDX, ROW), jnp.float32) for _ in range(2)],
        [pltpu.VMEM((CHUNK_IDX,), jnp.float32) for _ in range(2)],
        [pltpu.SemaphoreType.DMA for _ in range(2)],
        [pltpu.SemaphoreType.DMA for _ in range(2)],
        [pltpu.SemaphoreType.DMA for _ in range(2)],
        pltpu.VMEM_SHARED((TABLE_ROWS, ROW), jnp.float32),
    ],
    compiler_params=pltpu.CompilerParams(
        use_tc_tiling_on_sc=False, needs_layout_passes=False),
)
def _sc_dist(coords_hbm, nbr_hbm, out_hbm, idx_vs, own_v, rbufs, out_vs,
             gsems, isems, osems, table_sh):
    sid = lax.axis_index("s")
    wid = sid * NC + lax.axis_index("c")
    pltpu.sync_copy(coords_hbm.at[pl.ds(sid * STAGE_ROWS, STAGE_ROWS)],
                    table_sh.at[pl.ds(sid * STAGE_ROWS, STAGE_ROWS)])
    plsc.subcore_barrier()

    start_atom = (wid * (N_ATOMS - ATOMS_PER_W)) // (NWORKERS - 1)

    iota = lax.iota(jnp.int32, LANES)
    col_x = jnp.zeros((LANES,), jnp.int32)
    col_y = col_x + 1
    col_z = col_x + 2

    def fire_half(idx_ref, h, buf, sem):
        pltpu.async_copy(table_sh.at[idx_ref.at[pl.ds(h * HALF_IDX,
                                                      HALF_IDX)]],
                         buf, sem)

    def drain_gather(buf, sem):
        pltpu.make_async_copy(
            coords_hbm.at[pl.ds(0, HALF_IDX)], buf, sem).wait()

    def compute_half(h, buf, off, out_v):
        @pl.loop(0, HALF_ATOMS)
        def _atom(a):
            own_row = own_v[off + h * HALF_ATOMS + a]
            ox = jnp.broadcast_to(own_row[0], (LANES,))
            oy = jnp.broadcast_to(own_row[1], (LANES,))
            oz = jnp.broadcast_to(own_row[2], (LANES,))
            for j in range(M_NBRS // LANES):
                ridx = iota + (a * M_NBRS + j * LANES)
                dx = plsc.load_gather(buf, [ridx, col_x]) - ox
                dy = plsc.load_gather(buf, [ridx, col_y]) - oy
                dz = plsc.load_gather(buf, [ridx, col_z]) - oz
                out_v[pl.ds(h * HALF_IDX + a * M_NBRS + j * LANES, LANES)] = (
                    dx * dx + dy * dy + dz * dz)

    pltpu.sync_copy(nbr_hbm.at[pl.ds(start_atom * M_NBRS, CHUNK_IDX)],
                    idx_vs[0])
    fire_half(idx_vs[0], 0, rbufs[0], gsems[0])

    @pl.loop(0, CHUNKS)
    def _chunk(c):
        p = lax.rem(c, 2)
        base_atom = start_atom + c * CHUNK_ATOMS
        base_idx = base_atom * M_NBRS

        @pl.when(c + 1 < CHUNKS)
        def _prefetch_idx():
            for q in range(2):
                @pl.when(p == q)
                def _():
                    pltpu.async_copy(
                        nbr_hbm.at[pl.ds(base_idx + CHUNK_IDX, CHUNK_IDX)],
                        idx_vs[1 - q], isems[1 - q])

        @pl.when(c >= 2)
        def _drain_out():
            for q in range(2):
                @pl.when(p == q)
                def _():
                    pltpu.make_async_copy(
                        out_vs[q], out_hbm.at[pl.ds(0, CHUNK_IDX)],
                        osems[q]).wait()

        aligned_base = (base_atom // 8) * 8
        off = base_atom - aligned_base
        pltpu.sync_copy(table_sh.at[pl.ds(aligned_base, WIN)], own_v)

        for q in range(2):
            @pl.when(p == q)
            def _chunk_body():
                idx_ref = idx_vs[q]
                out_v = out_vs[q]
                for h in range(HALVES):
                    b = h % 2
                    if h + 1 < HALVES:
                        fire_half(idx_ref, h + 1, rbufs[1 - b], gsems[1 - b])
                    else:
                        @pl.when(c + 1 < CHUNKS)
                        def _fire_next_chunk():
                            pltpu.make_async_copy(
                                nbr_hbm.at[pl.ds(0, CHUNK_IDX)],
                                idx_vs[1 - q], isems[1 - q]).wait()
                            fire_half(idx_vs[1 - q], 0, rbufs[1 - b],
                                      gsems[1 - b])
                    drain_gather(rbufs[b], gsems[b])
                    compute_half(h, rbufs[b], off, out_v)
                pltpu.async_copy(out_v, out_hbm.at[pl.ds(base_idx, CHUNK_IDX)],
                                 osems[q])

    for q in range(2):
        pltpu.make_async_copy(
            out_vs[q], out_hbm.at[pl.ds(0, CHUNK_IDX)], osems[q]).wait()


def kernel(coords, nbr_list):
    coords16 = jnp.pad(coords.astype(jnp.float32),
                       ((0, TABLE_ROWS - N_ATOMS), (0, ROW - 3)))
    nbr = nbr_list.astype(jnp.int32).reshape(-1)
    out = _sc_dist(coords16, nbr)
    return out.reshape(N_ATOMS, M_NBRS)
